# baseline (speedup 1.0000x reference)
import functools

import jax
import jax.numpy as jnp
from jax import lax
from jax.experimental import pallas as pl
from jax.experimental.pallas import tpu as pltpu
from jax.experimental.pallas import tpu_sc as plsc

NC = 2
NT = 16
NW = NC * NT
CH = 128
N_TRASH = 16


def _sc_mesh():
    return plsc.VectorSubcoreMesh(core_axis_name="c", subcore_axis_name="s")


def _make_deg_kernel(n_nodes, epr):
    rpw = epr // NW
    zch = 640
    n_acc = NT * zch

    @functools.partial(
        pl.kernel,
        out_type=jax.ShapeDtypeStruct((NC, n_acc), jnp.float32),
        mesh=_sc_mesh(),
        scratch_types=[
            pltpu.VMEM((rpw, CH), jnp.int32),
            pltpu.VMEM((CH,), jnp.float32),
            pltpu.VMEM_SHARED((n_acc,), jnp.float32),
        ],
    )
    def deg_k(rows_ref, ones_ref, zeros_ref, out_ref, rowv, onesv, accd):
        c = lax.axis_index("c")
        s = lax.axis_index("s")
        wid = s * NC + c
        pltpu.sync_copy(zeros_ref, accd.at[pl.ds(s * zch, zch)])
        pltpu.sync_copy(ones_ref, onesv)
        pltpu.sync_copy(rows_ref.at[pl.ds(wid * rpw, rpw)], rowv)
        plsc.subcore_barrier()

        def step(j, carry):
            pltpu.sync_copy(onesv, accd.at[rowv.at[j]], add=True)
            return carry
        lax.fori_loop(0, rpw, step, 0)
        plsc.subcore_barrier()
        pltpu.sync_copy(accd.at[pl.ds(s * zch, zch)],
                        out_ref.at[c].at[pl.ds(s * zch, zch)])

    return deg_k


def _make_layer_kernel(n_nodes, dim, epr64):
    chl = 64
    nbuf = 4
    rpw = epr64 // NW
    seg_rows = 40
    n_seg = rpw // seg_rows
    n_acc = n_nodes + N_TRASH
    zr, zr_base = 632, 632 * (NT - 1)
    zr_last = n_acc - zr_base
    dr_last = n_nodes - zr_base

    @functools.partial(
        pl.kernel,
        out_type=jax.ShapeDtypeStruct((NC, n_nodes, dim), jnp.float32),
        mesh=_sc_mesh(),
        scratch_types=[
            pltpu.VMEM((seg_rows, chl), jnp.int32),
            pltpu.VMEM((seg_rows, chl), jnp.int32),
            [pltpu.VMEM((chl, dim), jnp.float32) for _ in range(nbuf)],
            [pltpu.SemaphoreType.DMA for _ in range(nbuf)],
            pltpu.VMEM_SHARED((n_acc, dim), jnp.float32),
        ],
    )
    def layer_k(cols_ref, rows_ref, h_ref, zeros_ref, out_ref,
                colv, rowv, gbufs, sems, acc):
        c = lax.axis_index("c")
        s = lax.axis_index("s")
        wid = s * NC + c
        @pl.when(s < NT - 1)
        def _():
            pltpu.sync_copy(zeros_ref.at[pl.ds(s * zr, zr)],
                            acc.at[pl.ds(s * zr, zr)])
        @pl.when(s == NT - 1)
        def _():
            pltpu.sync_copy(zeros_ref.at[pl.ds(zr_base, zr_last)],
                            acc.at[pl.ds(zr_base, zr_last)])
        plsc.subcore_barrier()

        for seg in range(n_seg):
            base = wid * rpw + seg * seg_rows
            pltpu.sync_copy(cols_ref.at[pl.ds(base, seg_rows)], colv)
            pltpu.sync_copy(rows_ref.at[pl.ds(base, seg_rows)], rowv)
            for b in range(nbuf):
                pltpu.async_copy(h_ref.at[colv.at[b]], gbufs[b], sems[b])

            def step(p, carry):
                for b in range(nbuf):
                    j = nbuf * p + b
                    pltpu.make_async_copy(h_ref.at[colv.at[j]], gbufs[b],
                                          sems[b]).wait()
                    pltpu.sync_copy(gbufs[b], acc.at[rowv.at[j]], add=True)
                    @pl.when(p < seg_rows // nbuf - 1)
                    def _():
                        pltpu.async_copy(h_ref.at[colv.at[j + nbuf]],
                                         gbufs[b], sems[b])
                return carry
            lax.fori_loop(0, seg_rows // nbuf, step, 0)
        plsc.subcore_barrier()
        @pl.when(s < NT - 1)
        def _():
            pltpu.sync_copy(acc.at[pl.ds(s * zr, zr)],
                            out_ref.at[c].at[pl.ds(s * zr, zr)])
        @pl.when(s == NT - 1)
        def _():
            pltpu.sync_copy(acc.at[pl.ds(zr_base, dr_last)],
                            out_ref.at[c].at[pl.ds(zr_base, dr_last)])

    return layer_k


def _dis_h0(degp_t, e_weight, br=1000):
    n, dim = e_weight.shape

    def body(dp, e, dis_ref, h0_ref):
        deg = dp[:, 0:1] + dp[:, 1:2] + 1.0
        dis = lax.rsqrt(jnp.maximum(deg, 1e-12))
        dis_ref[...] = dis
        h0_ref[...] = e[...] * dis

    return pl.pallas_call(
        body,
        grid=(n // br,),
        in_specs=[
            pl.BlockSpec((br, 2), lambda i: (i, 0)),
            pl.BlockSpec((br, dim), lambda i: (i, 0)),
        ],
        out_specs=[
            pl.BlockSpec((br, 1), lambda i: (i, 0)),
            pl.BlockSpec((br, dim), lambda i: (i, 0)),
        ],
        out_shape=[
            jax.ShapeDtypeStruct((n, 1), jnp.float32),
            jax.ShapeDtypeStruct((n, dim), jnp.float32),
        ],
    )(degp_t, e_weight)


def _hnext(partials, h, dis, br=1000):
    _, n, dim = partials.shape

    def body(p, hh, dis_ref, hn_ref):
        d = dis_ref[...]
        hn_ref[...] = (p[0] + p[1] + hh[...]) * (d * d)

    return pl.pallas_call(
        body,
        grid=(n // br,),
        in_specs=[
            pl.BlockSpec((2, br, dim), lambda i: (0, i, 0)),
            pl.BlockSpec((br, dim), lambda i: (i, 0)),
            pl.BlockSpec((br, 1), lambda i: (i, 0)),
        ],
        out_specs=pl.BlockSpec((br, dim), lambda i: (i, 0)),
        out_shape=jax.ShapeDtypeStruct((n, dim), jnp.float32),
    )(partials, h, dis)


def _finalize(p3, h2, h3, dis, br=1000):
    _, n, dim = p3.shape

    def body(p, h2_ref, h3_ref, dis_ref, af_ref, f_ref):
        d = dis_ref[...]
        inv = 1.0 / d
        f3 = (p[0] + p[1] + h3_ref[...]) * d
        f1 = h2_ref[...] * inv
        f2 = h3_ref[...] * inv
        af_ref[...] = jnp.stack([f3, f1, f2], axis=1)
        f_ref[...] = f3

    return pl.pallas_call(
        body,
        grid=(n // br,),
        in_specs=[
            pl.BlockSpec((2, br, dim), lambda i: (0, i, 0)),
            pl.BlockSpec((br, dim), lambda i: (i, 0)),
            pl.BlockSpec((br, dim), lambda i: (i, 0)),
            pl.BlockSpec((br, 1), lambda i: (i, 0)),
        ],
        out_specs=[
            pl.BlockSpec((br, 3, dim), lambda i: (i, 0, 0)),
            pl.BlockSpec((br, dim), lambda i: (i, 0)),
        ],
        out_shape=[
            jax.ShapeDtypeStruct((n, 3, dim), jnp.float32),
            jax.ShapeDtypeStruct((n, dim), jnp.float32),
        ],
    )(p3, h2, h3, dis)


def kernel(edge_index, edge_values, E_weight):
    del edge_values
    n, dim = E_weight.shape
    e = edge_index.shape[1]
    n_layers = 3

    row = edge_index[0].astype(jnp.int32)
    col = edge_index[1].astype(jnp.int32)

    epad = -e % (CH * NW * 8)
    epr = (e + epad) // CH
    ar = jnp.arange(epad, dtype=jnp.int32)
    row_p = jnp.concatenate([row, n + (ar % N_TRASH)])
    col_p = jnp.concatenate([col, ar % n])
    rows2d = row_p.reshape(epr, CH)
    epr64 = (e + epad) // 64
    rows64 = row_p.reshape(epr64, 64)
    cols64 = col_p.reshape(epr64, 64)

    ones_v = jnp.ones((CH,), jnp.float32)
    zeros1 = jnp.zeros((640,), jnp.float32)
    zeros2 = jnp.zeros((n + N_TRASH, dim), jnp.float32)

    degp = _make_deg_kernel(n, epr)(rows2d, ones_v, zeros1)
    dis, h = _dis_h0(jnp.transpose(degp[:, :n]), E_weight)

    layer_k = _make_layer_kernel(n, dim, epr64)
    h1 = h
    p1 = layer_k(cols64, rows64, h1, zeros2)
    h2 = _hnext(p1, h1, dis)
    p2 = layer_k(cols64, rows64, h2, zeros2)
    h3 = _hnext(p2, h2, dis)
    p3 = layer_k(cols64, rows64, h3, zeros2)
    all_feat, feat = _finalize(p3, h2, h3, dis)
    del n_layers
    return all_feat, feat


# speedup vs baseline: 1.0223x; 1.0223x over previous
import functools

import jax
import jax.numpy as jnp
from jax import lax
from jax.experimental import pallas as pl
from jax.experimental.pallas import tpu as pltpu
from jax.experimental.pallas import tpu_sc as plsc

NC = 2
NT = 16
NW = NC * NT
CH = 128
N_TRASH = 16


def _sc_mesh():
    return plsc.VectorSubcoreMesh(core_axis_name="c", subcore_axis_name="s")


def _make_deg_kernel(n_nodes, epr):
    rpw = epr // NW
    zch = 640
    n_acc = NT * zch

    @functools.partial(
        pl.kernel,
        out_type=jax.ShapeDtypeStruct((NC, n_acc), jnp.float32),
        mesh=_sc_mesh(),
        scratch_types=[
            pltpu.VMEM((rpw, CH), jnp.int32),
            pltpu.VMEM((CH,), jnp.float32),
            pltpu.VMEM_SHARED((n_acc,), jnp.float32),
        ],
    )
    def deg_k(rows_ref, ones_ref, zeros_ref, out_ref, rowv, onesv, accd):
        c = lax.axis_index("c")
        s = lax.axis_index("s")
        wid = s * NC + c
        pltpu.sync_copy(zeros_ref, accd.at[pl.ds(s * zch, zch)])
        pltpu.sync_copy(ones_ref, onesv)
        pltpu.sync_copy(rows_ref.at[pl.ds(wid * rpw, rpw)], rowv)
        plsc.subcore_barrier()

        def step(j, carry):
            pltpu.sync_copy(onesv, accd.at[rowv.at[j]], add=True)
            return carry
        lax.fori_loop(0, rpw, step, 0)
        plsc.subcore_barrier()
        pltpu.sync_copy(accd.at[pl.ds(s * zch, zch)],
                        out_ref.at[c].at[pl.ds(s * zch, zch)])

    return deg_k


def _make_layer_kernel(n_nodes, dim, epr64):
    chl = 64
    nbuf = 4
    rpw = epr64 // NW
    seg_rows = 40
    n_seg = rpw // seg_rows
    n_acc = n_nodes + N_TRASH
    zr, zr_base = 632, 632 * (NT - 1)
    zr_last = n_acc - zr_base
    dr_last = n_nodes - zr_base

    @functools.partial(
        pl.kernel,
        out_type=jax.ShapeDtypeStruct((NC, n_nodes, dim), jnp.float32),
        mesh=_sc_mesh(),
        scratch_types=[
            pltpu.VMEM((seg_rows, chl), jnp.int32),
            pltpu.VMEM((seg_rows, chl), jnp.int32),
            [pltpu.VMEM((chl, dim), jnp.float32) for _ in range(nbuf)],
            [pltpu.SemaphoreType.DMA for _ in range(nbuf)],
            pltpu.VMEM_SHARED((n_acc, dim), jnp.float32),
        ],
    )
    def layer_k(cols_ref, rows_ref, h_ref, zeros_ref, out_ref,
                colv, rowv, gbufs, sems, acc):
        c = lax.axis_index("c")
        s = lax.axis_index("s")
        wid = s * NC + c
        @pl.when(s < NT - 1)
        def _():
            pltpu.sync_copy(zeros_ref.at[pl.ds(s * zr, zr)],
                            acc.at[pl.ds(s * zr, zr)])
        @pl.when(s == NT - 1)
        def _():
            pltpu.sync_copy(zeros_ref.at[pl.ds(zr_base, zr_last)],
                            acc.at[pl.ds(zr_base, zr_last)])
        plsc.subcore_barrier()

        for seg in range(n_seg):
            base = wid * rpw + seg * seg_rows
            pltpu.sync_copy(cols_ref.at[pl.ds(base, seg_rows)], colv)
            pltpu.sync_copy(rows_ref.at[pl.ds(base, seg_rows)], rowv)
            for b in range(nbuf):
                pltpu.async_copy(h_ref.at[colv.at[b]], gbufs[b], sems[b])

            def step(p, carry):
                for b in range(nbuf):
                    j = nbuf * p + b
                    pltpu.make_async_copy(h_ref.at[colv.at[j]], gbufs[b],
                                          sems[b]).wait()
                    pltpu.sync_copy(gbufs[b], acc.at[rowv.at[j]], add=True)
                    @pl.when(p < seg_rows // nbuf - 1)
                    def _():
                        pltpu.async_copy(h_ref.at[colv.at[j + nbuf]],
                                         gbufs[b], sems[b])
                return carry
            lax.fori_loop(0, seg_rows // nbuf, step, 0)
        plsc.subcore_barrier()
        @pl.when(s < NT - 1)
        def _():
            pltpu.sync_copy(acc.at[pl.ds(s * zr, zr)],
                            out_ref.at[c].at[pl.ds(s * zr, zr)])
        @pl.when(s == NT - 1)
        def _():
            pltpu.sync_copy(acc.at[pl.ds(zr_base, dr_last)],
                            out_ref.at[c].at[pl.ds(zr_base, dr_last)])

    return layer_k


def _dis_h0(degp_t, e_weight, br=1000):
    n, dim = e_weight.shape

    def body(dp, e, dis_ref, h0_ref):
        deg = dp[:, 0:1] + dp[:, 1:2] + 1.0
        dis = lax.rsqrt(jnp.maximum(deg, 1e-12))
        dis_ref[...] = dis
        h0_ref[...] = e[...] * dis

    return pl.pallas_call(
        body,
        grid=(n // br,),
        in_specs=[
            pl.BlockSpec((br, 2), lambda i: (i, 0)),
            pl.BlockSpec((br, dim), lambda i: (i, 0)),
        ],
        out_specs=[
            pl.BlockSpec((br, 1), lambda i: (i, 0)),
            pl.BlockSpec((br, dim), lambda i: (i, 0)),
        ],
        out_shape=[
            jax.ShapeDtypeStruct((n, 1), jnp.float32),
            jax.ShapeDtypeStruct((n, dim), jnp.float32),
        ],
    )(degp_t, e_weight)


def _combine(partials, h, dis, br=1000):
    _, n, dim = partials.shape

    def body(p, hh, dis_ref, f_ref, hn_ref):
        d = dis_ref[...]
        f = (p[0] + p[1] + hh[...]) * d
        f_ref[...] = f
        hn_ref[...] = f * d

    return pl.pallas_call(
        body,
        grid=(n // br,),
        in_specs=[
            pl.BlockSpec((2, br, dim), lambda i: (0, i, 0)),
            pl.BlockSpec((br, dim), lambda i: (i, 0)),
            pl.BlockSpec((br, 1), lambda i: (i, 0)),
        ],
        out_specs=[
            pl.BlockSpec((br, dim), lambda i: (i, 0)),
            pl.BlockSpec((br, dim), lambda i: (i, 0)),
        ],
        out_shape=[
            jax.ShapeDtypeStruct((n, dim), jnp.float32),
            jax.ShapeDtypeStruct((n, dim), jnp.float32),
        ],
    )(partials, h, dis)


def kernel(edge_index, edge_values, E_weight):
    del edge_values
    n, dim = E_weight.shape
    e = edge_index.shape[1]
    n_layers = 3

    row = edge_index[0].astype(jnp.int32)
    col = edge_index[1].astype(jnp.int32)

    epad = -e % (CH * NW * 8)
    epr = (e + epad) // CH
    ar = jnp.arange(epad, dtype=jnp.int32)
    row_p = jnp.concatenate([row, n + (ar % N_TRASH)])
    col_p = jnp.concatenate([col, ar % n])
    rows2d = row_p.reshape(epr, CH)
    epr64 = (e + epad) // 64
    rows64 = row_p.reshape(epr64, 64)
    cols64 = col_p.reshape(epr64, 64)

    ones_v = jnp.ones((CH,), jnp.float32)
    zeros1 = jnp.zeros((640,), jnp.float32)
    zeros2 = jnp.zeros((n + N_TRASH, dim), jnp.float32)

    degp = _make_deg_kernel(n, epr)(rows2d, ones_v, zeros1)
    dis, h = _dis_h0(jnp.transpose(degp[:, :n]), E_weight)

    layer_k = _make_layer_kernel(n, dim, epr64)
    feats = []
    for _ in range(n_layers):
        partials = layer_k(cols64, rows64, h, zeros2)
        feat, h = _combine(partials, h, dis)
        feats.append(feat)

    all_feat = jnp.stack([feats[2], feats[0], feats[1]], axis=1)
    return all_feat, feats[2]


# speedup vs baseline: 1.0387x; 1.0160x over previous
import functools

import jax
import jax.numpy as jnp
from jax import lax
from jax.experimental import pallas as pl
from jax.experimental.pallas import tpu as pltpu
from jax.experimental.pallas import tpu_sc as plsc

NC = 2
NT = 16
NW = NC * NT
CH = 128
N_TRASH = 16


def _sc_mesh():
    return plsc.VectorSubcoreMesh(core_axis_name="c", subcore_axis_name="s")


def _make_deg_kernel(n_nodes, epr):
    rpw = epr // NW
    zch = 640
    n_acc = NT * zch

    @functools.partial(
        pl.kernel,
        out_type=jax.ShapeDtypeStruct((NC, n_acc), jnp.float32),
        mesh=_sc_mesh(),
        scratch_types=[
            pltpu.VMEM((rpw, CH), jnp.int32),
            pltpu.VMEM((CH,), jnp.float32),
            pltpu.VMEM_SHARED((n_acc,), jnp.float32),
        ],
    )
    def deg_k(rows_ref, ones_ref, zeros_ref, out_ref, rowv, onesv, accd):
        c = lax.axis_index("c")
        s = lax.axis_index("s")
        wid = s * NC + c
        pltpu.sync_copy(zeros_ref, accd.at[pl.ds(s * zch, zch)])
        pltpu.sync_copy(ones_ref, onesv)
        pltpu.sync_copy(rows_ref.at[pl.ds(wid * rpw, rpw)], rowv)
        plsc.subcore_barrier()

        def step(j, carry):
            pltpu.sync_copy(onesv, accd.at[rowv.at[j]], add=True)
            return carry
        lax.fori_loop(0, rpw, step, 0)
        plsc.subcore_barrier()
        pltpu.sync_copy(accd.at[pl.ds(s * zch, zch)],
                        out_ref.at[c].at[pl.ds(s * zch, zch)])

    return deg_k


def _make_layer_kernel(n_nodes, dim, epr64):
    chl = 64
    nbuf = 4
    rpw = epr64 // NW
    seg_rows = 40
    n_seg = rpw // seg_rows
    n_acc = n_nodes + N_TRASH
    zr, zr_base = 632, 632 * (NT - 1)
    zr_last = n_acc - zr_base
    dr_last = n_nodes - zr_base

    @functools.partial(
        pl.kernel,
        out_type=jax.ShapeDtypeStruct((NC, n_nodes, dim), jnp.float32),
        mesh=_sc_mesh(),
        scratch_types=[
            pltpu.VMEM((seg_rows, chl), jnp.int32),
            pltpu.VMEM((seg_rows, chl), jnp.int32),
            [pltpu.VMEM((chl, dim), jnp.float32) for _ in range(nbuf)],
            [pltpu.SemaphoreType.DMA for _ in range(nbuf)],
            pltpu.VMEM_SHARED((n_acc, dim), jnp.float32),
        ],
    )
    def layer_k(cols_ref, rows_ref, h_ref, zeros_ref, out_ref,
                colv, rowv, gbufs, sems, acc):
        c = lax.axis_index("c")
        s = lax.axis_index("s")
        wid = s * NC + c
        @pl.when(s < NT - 1)
        def _():
            pltpu.sync_copy(zeros_ref.at[pl.ds(s * zr, zr)],
                            acc.at[pl.ds(s * zr, zr)])
        @pl.when(s == NT - 1)
        def _():
            pltpu.sync_copy(zeros_ref.at[pl.ds(zr_base, zr_last)],
                            acc.at[pl.ds(zr_base, zr_last)])
        plsc.subcore_barrier()

        for seg in range(n_seg):
            base = wid * rpw + seg * seg_rows
            pltpu.sync_copy(cols_ref.at[pl.ds(base, seg_rows)], colv)
            pltpu.sync_copy(rows_ref.at[pl.ds(base, seg_rows)], rowv)
            for b in range(nbuf):
                pltpu.async_copy(h_ref.at[colv.at[b]], gbufs[b], sems[b])

            def step(p, carry):
                for b in range(nbuf):
                    j = nbuf * p + b
                    pltpu.make_async_copy(h_ref.at[colv.at[j]], gbufs[b],
                                          sems[b]).wait()
                    pltpu.sync_copy(gbufs[b], acc.at[rowv.at[j]], add=True)
                    @pl.when(p < seg_rows // nbuf - 1)
                    def _():
                        pltpu.async_copy(h_ref.at[colv.at[j + nbuf]],
                                         gbufs[b], sems[b])
                return carry
            lax.fori_loop(0, seg_rows // nbuf, step, 0)
        plsc.subcore_barrier()
        @pl.when(s < NT - 1)
        def _():
            pltpu.sync_copy(acc.at[pl.ds(s * zr, zr)],
                            out_ref.at[c].at[pl.ds(s * zr, zr)])
        @pl.when(s == NT - 1)
        def _():
            pltpu.sync_copy(acc.at[pl.ds(zr_base, dr_last)],
                            out_ref.at[c].at[pl.ds(zr_base, dr_last)])

    return layer_k


def _dis_h0(degp_t, e_weight, br=2000):
    n, dim = e_weight.shape

    def body(dp, e, dis_ref, h0_ref):
        deg = dp[:, 0:1] + dp[:, 1:2] + 1.0
        dis = lax.rsqrt(jnp.maximum(deg, 1e-12))
        dis_ref[...] = dis
        h0_ref[...] = e[...] * dis

    return pl.pallas_call(
        body,
        grid=(n // br,),
        in_specs=[
            pl.BlockSpec((br, 2), lambda i: (i, 0)),
            pl.BlockSpec((br, dim), lambda i: (i, 0)),
        ],
        out_specs=[
            pl.BlockSpec((br, 1), lambda i: (i, 0)),
            pl.BlockSpec((br, dim), lambda i: (i, 0)),
        ],
        out_shape=[
            jax.ShapeDtypeStruct((n, 1), jnp.float32),
            jax.ShapeDtypeStruct((n, dim), jnp.float32),
        ],
    )(degp_t, e_weight)


def _combine(partials, h, dis, br=2000):
    _, n, dim = partials.shape

    def body(p, hh, dis_ref, f_ref, hn_ref):
        d = dis_ref[...]
        f = (p[0] + p[1] + hh[...]) * d
        f_ref[...] = f
        hn_ref[...] = f * d

    return pl.pallas_call(
        body,
        grid=(n // br,),
        in_specs=[
            pl.BlockSpec((2, br, dim), lambda i: (0, i, 0)),
            pl.BlockSpec((br, dim), lambda i: (i, 0)),
            pl.BlockSpec((br, 1), lambda i: (i, 0)),
        ],
        out_specs=[
            pl.BlockSpec((br, dim), lambda i: (i, 0)),
            pl.BlockSpec((br, dim), lambda i: (i, 0)),
        ],
        out_shape=[
            jax.ShapeDtypeStruct((n, dim), jnp.float32),
            jax.ShapeDtypeStruct((n, dim), jnp.float32),
        ],
    )(partials, h, dis)


def kernel(edge_index, edge_values, E_weight):
    del edge_values
    n, dim = E_weight.shape
    e = edge_index.shape[1]
    n_layers = 3

    row = edge_index[0].astype(jnp.int32)
    col = edge_index[1].astype(jnp.int32)

    epad = -e % (CH * NW * 8)
    epr = (e + epad) // CH
    ar = jnp.arange(epad, dtype=jnp.int32)
    row_p = jnp.concatenate([row, n + (ar % N_TRASH)])
    col_p = jnp.concatenate([col, ar % n])
    rows2d = row_p.reshape(epr, CH)
    epr64 = (e + epad) // 64
    rows64 = row_p.reshape(epr64, 64)
    cols64 = col_p.reshape(epr64, 64)

    ones_v = jnp.ones((CH,), jnp.float32)
    zeros1 = jnp.zeros((640,), jnp.float32)
    zeros2 = jnp.zeros((n + N_TRASH, dim), jnp.float32)

    degp = _make_deg_kernel(n, epr)(rows2d, ones_v, zeros1)
    dis, h = _dis_h0(jnp.transpose(degp[:, :n]), E_weight)

    layer_k = _make_layer_kernel(n, dim, epr64)
    feats = []
    for _ in range(n_layers):
        partials = layer_k(cols64, rows64, h, zeros2)
        feat, h = _combine(partials, h, dis)
        feats.append(feat)

    all_feat = jnp.stack([feats[2], feats[0], feats[1]], axis=1)
    return all_feat, feats[2]
